# baseline (speedup 1.0000x reference)
import jax
import jax.numpy as jnp
from jax import lax
from jax.experimental import pallas as pl
from jax.experimental.pallas import tpu as pltpu
from jax.experimental.pallas import tpu_sc as plsc

_B = 16384
_NW = 32
_BPW = _B // _NW
_PW = 128
_HB = _BPW // 2


def _run_tasks(base, tasks, bufs, sem):
  copies = [None] * len(tasks)
  for k in (0, 1):
    tab, idx_v, _, h = tasks[k]
    copies[k] = pltpu.async_copy(tab.at[idx_v.at[pl.ds(h * _HB, _HB)]],
                                 bufs[k % 2], sem)
  for k, (tab, idx_v, out, h) in enumerate(tasks):
    copies[k].wait()
    pltpu.sync_copy(bufs[k % 2], out.at[pl.ds(base + h * _HB, _HB)])
    if k + 2 < len(tasks):
      tab2, idx2, _, h2 = tasks[k + 2]
      copies[k + 2] = pltpu.async_copy(
          tab2.at[idx2.at[pl.ds(h2 * _HB, _HB)]], bufs[k % 2], sem)


def _sc_small_body(art_id_h, gen_id_h, lang_id_h, art_t, gen_t, lang_t,
                   art_o, gen_o, lang_o, aidx, gidx, lidx,
                   rows_a, rows_b, sem):
  wid = lax.axis_index("s") * 2 + lax.axis_index("c")
  base = wid * _BPW
  pltpu.sync_copy(art_id_h.at[pl.ds(base, _BPW)], aidx)
  pltpu.sync_copy(gen_id_h.at[pl.ds(base, _BPW)], gidx)
  pltpu.sync_copy(lang_id_h.at[pl.ds(base, _BPW)], lidx)
  tasks = []
  for tab, idx_v, out in ((art_t, aidx, art_o), (gen_t, gidx, gen_o),
                          (lang_t, lidx, lang_o)):
    for h in (0, 1):
      tasks.append((tab, idx_v, out, h))
  _run_tasks(base, tasks, (rows_a, rows_b), sem)


def _sc_song_body(song_id_h, song_t, song_o, sidx, rows_a, rows_b, sem):
  wid = lax.axis_index("s") * 2 + lax.axis_index("c")
  base = wid * _BPW
  pltpu.sync_copy(song_id_h.at[pl.ds(base, _BPW)], sidx)
  tasks = [(song_t, sidx, song_o, 0), (song_t, sidx, song_o, 1)]
  _run_tasks(base, tasks, (rows_a, rows_b), sem)


_sc_gather_small = pl.kernel(
    _sc_small_body,
    out_type=[jax.ShapeDtypeStruct((_B, _PW), jnp.float32)] * 3,
    mesh=plsc.VectorSubcoreMesh(core_axis_name="c", subcore_axis_name="s"),
    scratch_types=[
        pltpu.VMEM((_BPW,), jnp.int32),
        pltpu.VMEM((_BPW,), jnp.int32),
        pltpu.VMEM((_BPW,), jnp.int32),
        pltpu.VMEM((_HB, _PW), jnp.float32),
        pltpu.VMEM((_HB, _PW), jnp.float32),
        pltpu.SemaphoreType.DMA,
    ],
    compiler_params=pltpu.CompilerParams(use_tc_tiling_on_sc=True),
    name="sc_gather_small",
)


_sc_gather_song = pl.kernel(
    _sc_song_body,
    out_type=jax.ShapeDtypeStruct((_B, _PW), jnp.float32),
    mesh=plsc.VectorSubcoreMesh(core_axis_name="c", subcore_axis_name="s"),
    scratch_types=[
        pltpu.VMEM((_BPW,), jnp.int32),
        pltpu.VMEM((_HB, _PW), jnp.float32),
        pltpu.VMEM((_HB, _PW), jnp.float32),
        pltpu.SemaphoreType.DMA,
    ],
    compiler_params=pltpu.CompilerParams(use_tc_tiling_on_sc=True),
    name="sc_gather_song",
)


_PC = 16384


def _pack_body(xt, out):
  t = jnp.transpose(xt[...], (1, 0))
  t2 = t.reshape(_PC // 2, 2, 64)
  out[...] = jnp.concatenate([t2[:, 0, :], t2[:, 1, :]], axis=1)


def _pack_pairs(table):
  vocab = table.shape[0]
  nsteps = (vocab + _PC - 1) // _PC
  return pl.pallas_call(
      _pack_body,
      grid=(nsteps,),
      in_specs=[pl.BlockSpec((64, _PC), lambda i: (0, i))],
      out_specs=pl.BlockSpec((_PC // 2, 128), lambda i: (i, 0)),
      out_shape=jax.ShapeDtypeStruct((vocab // 2, 128), jnp.float32),
  )(table.T)


def _unpack(pairs, hsel):
  return jnp.where(hsel > 0.5, pairs[:, 64:128], pairs[:, 0:64])


_CHUNK = 2048


def _mlp_body(song_q, art_q, gen, lang, num, ssel, asel,
              w1a, w1b, w1c, w1d, wnum, bnum, w1e, b1, w2, b2, w3, b3, out):
  song = _unpack(song_q[...], ssel[:, 0:1])
  art = _unpack(art_q[...], asel[:, 0:1])
  acc = jnp.dot(song, w1a[...], preferred_element_type=jnp.float32)
  acc += jnp.dot(art, w1b[...], preferred_element_type=jnp.float32)
  acc += jnp.dot(gen[...], w1c[...], preferred_element_type=jnp.float32)
  acc += jnp.dot(lang[...], w1d[...], preferred_element_type=jnp.float32)
  nv = jnp.dot(num[...], wnum[...], preferred_element_type=jnp.float32) + bnum[...]
  acc += jnp.dot(nv, w1e[...], preferred_element_type=jnp.float32)
  h1 = jnp.maximum(acc + b1[...], 0.0)
  h2 = jnp.maximum(jnp.dot(h1, w2[...], preferred_element_type=jnp.float32) + b2[...], 0.0)
  out[...] = jnp.dot(h2, w3[...], preferred_element_type=jnp.float32) + b3[...]


def _tc_mlp(song_q, art_q, gen_emb, lang_emb, num, ssel, asel,
            w1a, w1b, w1c, w1d, wnum, bnum, w1e, b1, w2, b2, w3, b3):
  nsteps = _B // _CHUNK
  row_spec = lambda width: pl.BlockSpec((_CHUNK, width), lambda i: (i, 0))
  full = lambda a: pl.BlockSpec(a.shape, lambda i: (0,) * a.ndim)
  return pl.pallas_call(
      _mlp_body,
      grid=(nsteps,),
      in_specs=[
          row_spec(_PW), row_spec(_PW), row_spec(_PW), row_spec(_PW),
          row_spec(8), row_spec(2), row_spec(2),
          full(w1a), full(w1b), full(w1c), full(w1d), full(wnum), full(bnum),
          full(w1e), full(b1), full(w2), full(b2), full(w3), full(b3),
      ],
      out_specs=pl.BlockSpec((_CHUNK, 64), lambda i: (i, 0)),
      out_shape=jax.ShapeDtypeStruct((_B, 64), jnp.float32),
  )(song_q, art_q, gen_emb, lang_emb, num, ssel, asel,
    w1a, w1b, w1c, w1d, wnum, bnum, w1e, b1, w2, b2, w3, b3)


def _sel(ids):
  s = ids % 2
  return jnp.stack([s, s], axis=1).astype(jnp.float32)


def kernel(song_id, artist_encoded, genre_encoded, language_encoded,
           numerical_features, song_table, artist_table, genre_table,
           language_table, W_num, b_num, W1, b1, W2, b2, W3, b3):
  song_id = song_id.astype(jnp.int32)
  artist_encoded = artist_encoded.astype(jnp.int32)
  art_p = _pack_pairs(artist_table)
  gen_p = jnp.pad(genre_table, ((0, 0), (0, _PW - 31)))
  lang_p = jnp.pad(language_table, ((0, 0), (0, _PW - 31)))
  art_q, gen_emb, lang_emb = _sc_gather_small(
      artist_encoded // 2,
      genre_encoded.astype(jnp.int32), language_encoded.astype(jnp.int32),
      art_p, gen_p, lang_p)
  song_p = _pack_pairs(song_table)
  song_q = _sc_gather_song(song_id // 2, song_p)
  w1a = W1[0:64]
  w1b = W1[64:128]
  w1c = jnp.pad(W1[128:159], ((0, _PW - 31), (0, 0)))
  w1d = jnp.pad(W1[159:190], ((0, _PW - 31), (0, 0)))
  w1e = W1[190:206]
  return _tc_mlp(song_q, art_q, gen_emb, lang_emb, numerical_features,
                 _sel(song_id), _sel(artist_encoded),
                 w1a, w1b, w1c, w1d, W_num, b_num.reshape(1, 16), w1e,
                 b1.reshape(1, 256), W2, b2.reshape(1, 128), W3,
                 b3.reshape(1, 64))


# speedup vs baseline: 1.0758x; 1.0758x over previous
import jax
import jax.numpy as jnp
from jax import lax
from jax.experimental import pallas as pl
from jax.experimental.pallas import tpu as pltpu
from jax.experimental.pallas import tpu_sc as plsc

_B = 16384
_NW = 32
_BPW = _B // _NW
_PW = 128
_HB = _BPW // 2


def _run_tasks(base, tasks, bufs, sem):
  copies = [None] * len(tasks)
  for k in (0, 1):
    tab, idx_v, _, h = tasks[k]
    copies[k] = pltpu.async_copy(tab.at[idx_v.at[pl.ds(h * _HB, _HB)]],
                                 bufs[k % 2], sem)
  for k, (tab, idx_v, out, h) in enumerate(tasks):
    copies[k].wait()
    pltpu.sync_copy(bufs[k % 2], out.at[pl.ds(base + h * _HB, _HB)])
    if k + 2 < len(tasks):
      tab2, idx2, _, h2 = tasks[k + 2]
      copies[k + 2] = pltpu.async_copy(
          tab2.at[idx2.at[pl.ds(h2 * _HB, _HB)]], bufs[k % 2], sem)


def _sc_small_body(art_id_h, gen_id_h, lang_id_h, art_t, gen_t, lang_t,
                   art_o, gen_o, lang_o, aidx, gidx, lidx,
                   rows_a, rows_b, sem):
  wid = lax.axis_index("s") * 2 + lax.axis_index("c")
  base = wid * _BPW
  pltpu.sync_copy(art_id_h.at[pl.ds(base, _BPW)], aidx)
  pltpu.sync_copy(gen_id_h.at[pl.ds(base, _BPW)], gidx)
  pltpu.sync_copy(lang_id_h.at[pl.ds(base, _BPW)], lidx)
  tasks = []
  for tab, idx_v, out in ((art_t, aidx, art_o), (gen_t, gidx, gen_o),
                          (lang_t, lidx, lang_o)):
    for h in (0, 1):
      tasks.append((tab, idx_v, out, h))
  _run_tasks(base, tasks, (rows_a, rows_b), sem)


def _sc_song_body(song_id_h, song_t, song_o, sidx, rows_a, rows_b, sem):
  wid = lax.axis_index("s") * 2 + lax.axis_index("c")
  base = wid * _BPW
  pltpu.sync_copy(song_id_h.at[pl.ds(base, _BPW)], sidx)
  tasks = [(song_t, sidx, song_o, 0), (song_t, sidx, song_o, 1)]
  _run_tasks(base, tasks, (rows_a, rows_b), sem)


_sc_gather_small = pl.kernel(
    _sc_small_body,
    out_type=[jax.ShapeDtypeStruct((_B, _PW), jnp.uint32)] * 3,
    mesh=plsc.VectorSubcoreMesh(core_axis_name="c", subcore_axis_name="s"),
    scratch_types=[
        pltpu.VMEM((_BPW,), jnp.int32),
        pltpu.VMEM((_BPW,), jnp.int32),
        pltpu.VMEM((_BPW,), jnp.int32),
        pltpu.VMEM((_HB, _PW), jnp.uint32),
        pltpu.VMEM((_HB, _PW), jnp.uint32),
        pltpu.SemaphoreType.DMA,
    ],
    compiler_params=pltpu.CompilerParams(use_tc_tiling_on_sc=True),
    name="sc_gather_small",
)


_sc_gather_song = pl.kernel(
    _sc_song_body,
    out_type=jax.ShapeDtypeStruct((_B, _PW), jnp.uint32),
    mesh=plsc.VectorSubcoreMesh(core_axis_name="c", subcore_axis_name="s"),
    scratch_types=[
        pltpu.VMEM((_BPW,), jnp.int32),
        pltpu.VMEM((_HB, _PW), jnp.uint32),
        pltpu.VMEM((_HB, _PW), jnp.uint32),
        pltpu.SemaphoreType.DMA,
    ],
    compiler_params=pltpu.CompilerParams(use_tc_tiling_on_sc=True),
    name="sc_gather_song",
)


_PC = 16384


def _pack_body(xt, out):
  xb = xt[...].astype(jnp.bfloat16)
  t = jnp.transpose(xb, (1, 0))
  t4 = t.reshape(_PC // 4, 4, 64)

  def u(k):
    return lax.bitcast_convert_type(
        t4[:, k, :], jnp.uint16).astype(jnp.uint32)

  w1 = u(0) | (u(1) << 16)
  w2 = u(2) | (u(3) << 16)
  out[...] = jnp.concatenate([w1, w2], axis=1)


def _pack_quads(table):
  vocab = table.shape[0]
  nsteps = (vocab + _PC - 1) // _PC
  return pl.pallas_call(
      _pack_body,
      grid=(nsteps,),
      in_specs=[pl.BlockSpec((64, _PC), lambda i: (0, i))],
      out_specs=pl.BlockSpec((_PC // 4, 128), lambda i: (i, 0)),
      out_shape=jax.ShapeDtypeStruct((vocab // 4, 128), jnp.uint32),
  )(table.T)


def _unpack(quads, hsel, parity):
  w = jnp.where(hsel > 0.5, quads[:, 64:128], quads[:, 0:64])
  lo = lax.bitcast_convert_type(w << 16, jnp.float32)
  hi = lax.bitcast_convert_type(w & jnp.uint32(0xFFFF0000), jnp.float32)
  return jnp.where(parity > 0.5, hi, lo)


_CHUNK = 2048


def _mlp_body(song_q, art_q, gen, lang, num, ssel, asel,
              w1a, w1b, w1c, w1d, wnum, bnum, w1e, b1, w2, b2, w3, b3, out):
  song = _unpack(song_q[...], ssel[:, 0:1], ssel[:, 1:2])
  art = _unpack(art_q[...], asel[:, 0:1], asel[:, 1:2])
  acc = jnp.dot(song, w1a[...], preferred_element_type=jnp.float32)
  acc += jnp.dot(art, w1b[...], preferred_element_type=jnp.float32)
  acc += jnp.dot(gen[...], w1c[...], preferred_element_type=jnp.float32)
  acc += jnp.dot(lang[...], w1d[...], preferred_element_type=jnp.float32)
  nv = jnp.dot(num[...], wnum[...], preferred_element_type=jnp.float32) + bnum[...]
  acc += jnp.dot(nv, w1e[...], preferred_element_type=jnp.float32)
  h1 = jnp.maximum(acc + b1[...], 0.0)
  h2 = jnp.maximum(jnp.dot(h1, w2[...], preferred_element_type=jnp.float32) + b2[...], 0.0)
  out[...] = jnp.dot(h2, w3[...], preferred_element_type=jnp.float32) + b3[...]


def _tc_mlp(song_q, art_q, gen_emb, lang_emb, num, ssel, asel,
            w1a, w1b, w1c, w1d, wnum, bnum, w1e, b1, w2, b2, w3, b3):
  nsteps = _B // _CHUNK
  row_spec = lambda width: pl.BlockSpec((_CHUNK, width), lambda i: (i, 0))
  full = lambda a: pl.BlockSpec(a.shape, lambda i: (0,) * a.ndim)
  return pl.pallas_call(
      _mlp_body,
      grid=(nsteps,),
      in_specs=[
          row_spec(_PW), row_spec(_PW), row_spec(_PW), row_spec(_PW),
          row_spec(8), row_spec(2), row_spec(2),
          full(w1a), full(w1b), full(w1c), full(w1d), full(wnum), full(bnum),
          full(w1e), full(b1), full(w2), full(b2), full(w3), full(b3),
      ],
      out_specs=pl.BlockSpec((_CHUNK, 64), lambda i: (i, 0)),
      out_shape=jax.ShapeDtypeStruct((_B, 64), jnp.float32),
  )(song_q, art_q, gen_emb, lang_emb, num, ssel, asel,
    w1a, w1b, w1c, w1d, wnum, bnum, w1e, b1, w2, b2, w3, b3)


def _sel(ids):
  s = ids % 4
  return jnp.stack([s // 2, s % 2], axis=1).astype(jnp.float32)


def kernel(song_id, artist_encoded, genre_encoded, language_encoded,
           numerical_features, song_table, artist_table, genre_table,
           language_table, W_num, b_num, W1, b1, W2, b2, W3, b3):
  song_id = song_id.astype(jnp.int32)
  artist_encoded = artist_encoded.astype(jnp.int32)
  art_p = _pack_quads(artist_table)
  gen_p = lax.bitcast_convert_type(
      jnp.pad(genre_table, ((0, 0), (0, _PW - 31))), jnp.uint32)
  lang_p = lax.bitcast_convert_type(
      jnp.pad(language_table, ((0, 0), (0, _PW - 31))), jnp.uint32)
  art_q, gen_u, lang_u = _sc_gather_small(
      artist_encoded // 4,
      genre_encoded.astype(jnp.int32), language_encoded.astype(jnp.int32),
      art_p, gen_p, lang_p)
  song_p = _pack_quads(song_table)
  song_q = _sc_gather_song(song_id // 4, song_p)
  gen_emb = lax.bitcast_convert_type(gen_u, jnp.float32)
  lang_emb = lax.bitcast_convert_type(lang_u, jnp.float32)
  w1a = W1[0:64]
  w1b = W1[64:128]
  w1c = jnp.pad(W1[128:159], ((0, _PW - 31), (0, 0)))
  w1d = jnp.pad(W1[159:190], ((0, _PW - 31), (0, 0)))
  w1e = W1[190:206]
  return _tc_mlp(song_q, art_q, gen_emb, lang_emb, numerical_features,
                 _sel(song_id), _sel(artist_encoded),
                 w1a, w1b, w1c, w1d, W_num, b_num.reshape(1, 16), w1e,
                 b1.reshape(1, 256), W2, b2.reshape(1, 128), W3,
                 b3.reshape(1, 64))


# speedup vs baseline: 1.1882x; 1.1044x over previous
import jax
import jax.numpy as jnp
from jax import lax
from jax.experimental import pallas as pl
from jax.experimental.pallas import tpu as pltpu
from jax.experimental.pallas import tpu_sc as plsc

_B = 16384
_NW = 32
_BPW = _B // _NW
_PW = 128
_HB = _BPW // 2


def _run_tasks(base, tasks, bufs, sem):
  copies = [None] * len(tasks)
  for k in (0, 1):
    tab, idx_v, _, h = tasks[k]
    copies[k] = pltpu.async_copy(tab.at[idx_v.at[pl.ds(h * _HB, _HB)]],
                                 bufs[k % 2], sem)
  for k, (tab, idx_v, out, h) in enumerate(tasks):
    copies[k].wait()
    pltpu.sync_copy(bufs[k % 2], out.at[pl.ds(base + h * _HB, _HB)])
    if k + 2 < len(tasks):
      tab2, idx2, _, h2 = tasks[k + 2]
      copies[k + 2] = pltpu.async_copy(
          tab2.at[idx2.at[pl.ds(h2 * _HB, _HB)]], bufs[k % 2], sem)


def _sc_small_body(art_id_h, gen_id_h, lang_id_h, art_t, gen_t, lang_t,
                   art_o, gen_o, lang_o, aidx, gidx, lidx,
                   rows_a, rows_b, sem):
  wid = lax.axis_index("s") * 2 + lax.axis_index("c")
  base = wid * _BPW
  pltpu.sync_copy(art_id_h.at[pl.ds(base, _BPW)], aidx)
  pltpu.sync_copy(gen_id_h.at[pl.ds(base, _BPW)], gidx)
  pltpu.sync_copy(lang_id_h.at[pl.ds(base, _BPW)], lidx)
  tasks = []
  for tab, idx_v, out in ((art_t, aidx, art_o), (gen_t, gidx, gen_o),
                          (lang_t, lidx, lang_o)):
    for h in (0, 1):
      tasks.append((tab, idx_v, out, h))
  _run_tasks(base, tasks, (rows_a, rows_b), sem)


def _sc_song_body(song_id_h, song_t, song_o, sidx, rows_a, rows_b, sem):
  wid = lax.axis_index("s") * 2 + lax.axis_index("c")
  base = wid * _BPW
  pltpu.sync_copy(song_id_h.at[pl.ds(base, _BPW)], sidx)
  tasks = [(song_t, sidx, song_o, 0), (song_t, sidx, song_o, 1)]
  _run_tasks(base, tasks, (rows_a, rows_b), sem)


_sc_gather_small = pl.kernel(
    _sc_small_body,
    out_type=[jax.ShapeDtypeStruct((_B, _PW), jnp.uint32)] * 3,
    mesh=plsc.VectorSubcoreMesh(core_axis_name="c", subcore_axis_name="s"),
    scratch_types=[
        pltpu.VMEM((_BPW,), jnp.int32),
        pltpu.VMEM((_BPW,), jnp.int32),
        pltpu.VMEM((_BPW,), jnp.int32),
        pltpu.VMEM((_HB, _PW), jnp.uint32),
        pltpu.VMEM((_HB, _PW), jnp.uint32),
        pltpu.SemaphoreType.DMA,
    ],
    compiler_params=pltpu.CompilerParams(use_tc_tiling_on_sc=True),
    name="sc_gather_small",
)


_sc_gather_song = pl.kernel(
    _sc_song_body,
    out_type=jax.ShapeDtypeStruct((_B, _PW), jnp.uint32),
    mesh=plsc.VectorSubcoreMesh(core_axis_name="c", subcore_axis_name="s"),
    scratch_types=[
        pltpu.VMEM((_BPW,), jnp.int32),
        pltpu.VMEM((_HB, _PW), jnp.uint32),
        pltpu.VMEM((_HB, _PW), jnp.uint32),
        pltpu.SemaphoreType.DMA,
    ],
    compiler_params=pltpu.CompilerParams(use_tc_tiling_on_sc=True),
    name="sc_gather_song",
)


_PC = 16384


def _pack_body(xt, out):
  xu = lax.bitcast_convert_type(xt[...], jnp.uint32)
  b = (xu + jnp.uint32(0x7FFF) + ((xu >> 16) & jnp.uint32(1))) >> 16
  t = jnp.transpose(b, (1, 0))
  t4 = t.reshape(_PC // 4, 4, 64)
  w1 = t4[:, 0, :] | (t4[:, 1, :] << 16)
  w2 = t4[:, 2, :] | (t4[:, 3, :] << 16)
  out[...] = jnp.concatenate([w1, w2], axis=1)


def _pack_quads(table):
  vocab = table.shape[0]
  nsteps = (vocab + _PC - 1) // _PC
  return pl.pallas_call(
      _pack_body,
      grid=(nsteps,),
      in_specs=[pl.BlockSpec((64, _PC), lambda i: (0, i))],
      out_specs=pl.BlockSpec((_PC // 4, 128), lambda i: (i, 0)),
      out_shape=jax.ShapeDtypeStruct((vocab // 4, 128), jnp.uint32),
  )(table.T)


def _unpack(quads, hsel, parity):
  w = jnp.where(hsel > 0.5, quads[:, 64:128], quads[:, 0:64])
  lo = lax.bitcast_convert_type(w << 16, jnp.float32)
  hi = lax.bitcast_convert_type(w & jnp.uint32(0xFFFF0000), jnp.float32)
  return jnp.where(parity > 0.5, hi, lo)


_CHUNK = 2048


def _mlp_body(song_q, art_q, gen, lang, num, ssel, asel,
              w1a, w1b, w1c, w1d, wnum, bnum, w1e, b1, w2, b2, w3, b3, out):
  song = _unpack(song_q[...], ssel[:, 0:1], ssel[:, 1:2])
  art = _unpack(art_q[...], asel[:, 0:1], asel[:, 1:2])
  acc = jnp.dot(song, w1a[...], preferred_element_type=jnp.float32)
  acc += jnp.dot(art, w1b[...], preferred_element_type=jnp.float32)
  acc += jnp.dot(gen[...], w1c[...], preferred_element_type=jnp.float32)
  acc += jnp.dot(lang[...], w1d[...], preferred_element_type=jnp.float32)
  nv = jnp.dot(num[...], wnum[...], preferred_element_type=jnp.float32) + bnum[...]
  acc += jnp.dot(nv, w1e[...], preferred_element_type=jnp.float32)
  h1 = jnp.maximum(acc + b1[...], 0.0)
  h2 = jnp.maximum(jnp.dot(h1, w2[...], preferred_element_type=jnp.float32) + b2[...], 0.0)
  out[...] = jnp.dot(h2, w3[...], preferred_element_type=jnp.float32) + b3[...]


def _tc_mlp(song_q, art_q, gen_emb, lang_emb, num, ssel, asel,
            w1a, w1b, w1c, w1d, wnum, bnum, w1e, b1, w2, b2, w3, b3):
  nsteps = _B // _CHUNK
  row_spec = lambda width: pl.BlockSpec((_CHUNK, width), lambda i: (i, 0))
  full = lambda a: pl.BlockSpec(a.shape, lambda i: (0,) * a.ndim)
  return pl.pallas_call(
      _mlp_body,
      grid=(nsteps,),
      in_specs=[
          row_spec(_PW), row_spec(_PW), row_spec(_PW), row_spec(_PW),
          row_spec(8), row_spec(2), row_spec(2),
          full(w1a), full(w1b), full(w1c), full(w1d), full(wnum), full(bnum),
          full(w1e), full(b1), full(w2), full(b2), full(w3), full(b3),
      ],
      out_specs=pl.BlockSpec((_CHUNK, 64), lambda i: (i, 0)),
      out_shape=jax.ShapeDtypeStruct((_B, 64), jnp.float32),
  )(song_q, art_q, gen_emb, lang_emb, num, ssel, asel,
    w1a, w1b, w1c, w1d, wnum, bnum, w1e, b1, w2, b2, w3, b3)


def _sel(ids):
  s = ids % 4
  return jnp.stack([s // 2, s % 2], axis=1).astype(jnp.float32)


def kernel(song_id, artist_encoded, genre_encoded, language_encoded,
           numerical_features, song_table, artist_table, genre_table,
           language_table, W_num, b_num, W1, b1, W2, b2, W3, b3):
  song_id = song_id.astype(jnp.int32)
  artist_encoded = artist_encoded.astype(jnp.int32)
  art_p = _pack_quads(artist_table)
  gen_p = lax.bitcast_convert_type(
      jnp.pad(genre_table, ((0, 0), (0, _PW - 31))), jnp.uint32)
  lang_p = lax.bitcast_convert_type(
      jnp.pad(language_table, ((0, 0), (0, _PW - 31))), jnp.uint32)
  art_q, gen_u, lang_u = _sc_gather_small(
      artist_encoded // 4,
      genre_encoded.astype(jnp.int32), language_encoded.astype(jnp.int32),
      art_p, gen_p, lang_p)
  song_p = _pack_quads(song_table)
  song_q = _sc_gather_song(song_id // 4, song_p)
  gen_emb = lax.bitcast_convert_type(gen_u, jnp.float32)
  lang_emb = lax.bitcast_convert_type(lang_u, jnp.float32)
  w1a = W1[0:64]
  w1b = W1[64:128]
  w1c = jnp.pad(W1[128:159], ((0, _PW - 31), (0, 0)))
  w1d = jnp.pad(W1[159:190], ((0, _PW - 31), (0, 0)))
  w1e = W1[190:206]
  return _tc_mlp(song_q, art_q, gen_emb, lang_emb, numerical_features,
                 _sel(song_id), _sel(artist_encoded),
                 w1a, w1b, w1c, w1d, W_num, b_num.reshape(1, 16), w1e,
                 b1.reshape(1, 256), W2, b2.reshape(1, 128), W3,
                 b3.reshape(1, 64))
